# Initial kernel scaffold; baseline (speedup 1.0000x reference)
import jax
import jax.numpy as jnp
from jax.experimental import pallas as pl


def kernel(x, edge_index, edge_weight, W1, b1, W2, b2, W_ih, W_hh, b_ih, b_hh, Wp, bp):
    raise NotImplementedError("write your pallas kernel here")


# speedup vs baseline: 11.6743x; 11.6743x over previous
import functools

import jax
import jax.numpy as jnp
import numpy as np
from jax import lax
from jax.experimental import pallas as pl
from jax.experimental.pallas import tpu as pltpu
from jax.experimental.pallas import tpu_sc as plsc

T = 8
N = 10000
E = 320000
D = 128
H1 = 64
HG = 32

NC = 2
NS = 16
NW = NC * NS
EPW = E // NW
SUB = 80
NSUB = EPW // SUB
STRIPE = 624
ZROWS = 104

_mesh = plsc.VectorSubcoreMesh(core_axis_name="c", subcore_axis_name="s")
_sc_params = pltpu.CompilerParams(needs_layout_passes=False,
                                  use_tc_tiling_on_sc=False)
_tc_params = pltpu.CompilerParams(vmem_limit_bytes=120 * 1024 * 1024)


@functools.partial(
    pl.kernel,
    mesh=_mesh,
    out_type=jax.ShapeDtypeStruct((T, NW, 1, N), jnp.float32),
    scratch_types=[
        pltpu.VMEM((1, N), jnp.float32),
        pltpu.VMEM((NSUB, SUB), jnp.int32),
        pltpu.VMEM((NSUB, SUB), jnp.float32),
    ],
    compiler_params=_sc_params,
)
def _deg_kernel(dst_hbm, w_hbm, out_hbm, deg_v, dst_v, w_v):
    c = lax.axis_index("c")
    s = lax.axis_index("s")
    gid = c * NS + s

    z16 = jnp.zeros((16,), jnp.float32)
    zsplat = jnp.zeros((16,), jnp.int32)
    lanes = lax.iota(jnp.int32, 16)
    lane_masks = [lanes == l for l in range(16)]

    def zero_body(i, carry):
        deg_v[0, pl.ds(i * 16, 16)] = z16
        return carry

    lax.fori_loop(0, N // 16, zero_body, 0)

    for t in range(T):
        pltpu.sync_copy(dst_hbm.at[t, gid], dst_v)
        pltpu.sync_copy(w_hbm.at[t, gid], w_v)

        def row_body(r, carry):
            for co in range(SUB // 16):
                d16 = dst_v[r, pl.ds(co * 16, 16)]
                w16 = w_v[r, pl.ds(co * 16, 16)]
                for l in range(16):
                    plsc.addupdate_scatter(deg_v, [zsplat, d16], w16,
                                           mask=lane_masks[l])
            return carry

        lax.fori_loop(0, NSUB, row_body, 0)

        pltpu.sync_copy(deg_v, out_hbm.at[t, gid])
        lax.fori_loop(0, N // 16, zero_body, 0)


@functools.partial(
    pl.kernel,
    mesh=_mesh,
    out_type=jax.ShapeDtypeStruct((NC, T, N, H1), jnp.float32),
    scratch_types=[
        pltpu.VMEM_SHARED((N, H1), jnp.float32),
        pltpu.VMEM((NSUB, SUB), jnp.int32),
        pltpu.VMEM((NSUB, SUB), jnp.int32),
        pltpu.VMEM((NSUB, SUB), jnp.float32),
        pltpu.VMEM((SUB, H1), jnp.float32),
        pltpu.VMEM((ZROWS, H1), jnp.float32),
        pltpu.SemaphoreType.DMA,
    ],
    compiler_params=_sc_params,
)
def _edge_kernel(y_hbm, src_hbm, dst_hbm, w_hbm, out_hbm,
                 acc, src_v, dst_v, w_v, rows_v, zero_v, sem):
    c = lax.axis_index("c")
    s = lax.axis_index("s")
    gid = c * NS + s
    base = s * STRIPE
    is_last = s == (NS - 1)

    z16 = jnp.zeros((16,), jnp.float32)
    zsplat = jnp.zeros((16,), jnp.int32)

    def zfill_body(i, carry):
        zero_v[i // 4, pl.ds((i % 4) * 16, 16)] = z16
        return carry

    lax.fori_loop(0, ZROWS * 4, zfill_body, 0)

    for t in range(T):
        for k in range(STRIPE // ZROWS):
            pltpu.sync_copy(zero_v, acc.at[pl.ds(base + k * ZROWS, ZROWS)])

        @pl.when(is_last)
        def _():
            pltpu.sync_copy(zero_v.at[pl.ds(0, 16)],
                            acc.at[pl.ds(N - 16, 16)])

        plsc.subcore_barrier()

        pltpu.sync_copy(src_hbm.at[t, gid], src_v)
        pltpu.sync_copy(dst_hbm.at[t, gid], dst_v)
        pltpu.sync_copy(w_hbm.at[t, gid], w_v)

        def sub_body(j, carry):
            pltpu.async_copy(y_hbm.at[src_v.at[j]], rows_v, sem).wait()

            def e_body(e, ecarry):
                wb = plsc.load_gather(w_v, [zsplat + j, zsplat + e])
                for rr in range(H1 // 16):
                    vals = rows_v[e, pl.ds(rr * 16, 16)]
                    rows_v[e, pl.ds(rr * 16, 16)] = vals * wb
                return ecarry

            lax.fori_loop(0, SUB, e_body, 0)

            pltpu.sync_copy(rows_v, acc.at[dst_v.at[j]], add=True)
            return carry

        lax.fori_loop(0, NSUB, sub_body, 0)
        plsc.subcore_barrier()

        pltpu.sync_copy(acc.at[pl.ds(base, STRIPE)],
                        out_hbm.at[c, t, pl.ds(base, STRIPE)])

        @pl.when(is_last)
        def _():
            pltpu.sync_copy(acc.at[pl.ds(N - 16, 16)],
                            out_hbm.at[c, t, pl.ds(N - 16, 16)])


_HIGH = lax.Precision.HIGHEST
NB = 5
BN = N // NB


def _tc1_body(x_ref, dp_ref, w1_ref, y1_ref, dis_ref):
    dp = dp_ref[0]
    ones64 = jnp.ones_like(dp[:, :H1])
    deg64 = lax.dot_general(dp, ones64, (((0,), (0,)), ((), ())),
                            precision=_HIGH) + 1.0
    dis64 = lax.rsqrt(deg64)
    xw = lax.dot_general(x_ref[0], w1_ref[...], (((0,), (0,)), ((), ())),
                         precision=_HIGH)
    y1_ref[0] = xw * dis64
    ones1 = jnp.ones_like(dp[:, :1])
    deg1 = lax.dot_general(dp, ones1, (((0,), (0,)), ((), ())),
                           precision=_HIGH) + 1.0
    dis_ref[0] = lax.rsqrt(deg1)


def _tc2_body(p_ref, y1_ref, dis_ref, w2_ref, b1_ref, y2_ref):
    S = p_ref[0, 0] + p_ref[1, 0]
    dis = dis_ref[0]
    h1 = jnp.maximum(dis * (S + y1_ref[0]) + b1_ref[...], 0.0)
    xw2 = lax.dot_general(h1, w2_ref[...], (((1,), (0,)), ((), ())),
                          precision=_HIGH)
    y2_ref[0] = xw2 * dis


def _tc3a_body(p_ref, y2_ref, dis_ref, b2_ref, g_ref):
    S = p_ref[0, 0] + p_ref[1, 0]
    dis = dis_ref[0]
    g_ref[0] = jnp.maximum(dis * (S + y2_ref[0]) + b2_ref[...], 0.0)


def _gru_body(g0_ref, g1_ref, g2_ref, g3_ref,
              wir_ref, wiz_ref, win_ref, whr_ref, whz_ref, whn_ref,
              bir_ref, biz_ref, bin_ref, bhr_ref, bhz_ref, bhn_ref,
              wp_ref, bp_ref, out_ref):
    t = pl.program_id(0)
    g_refs = (g0_ref, g1_ref, g2_ref, g3_ref)
    h = jnp.zeros_like(g0_ref[0][:, :HG])
    for k in range(4):
        valid = (t - 3 + k) >= 0
        xk = g_refs[k][0]
        i_r = lax.dot_general(xk, wir_ref[...], (((1,), (1,)), ((), ())),
                              precision=_HIGH) + bir_ref[...]
        i_z = lax.dot_general(xk, wiz_ref[...], (((1,), (1,)), ((), ())),
                              precision=_HIGH) + biz_ref[...]
        i_n = lax.dot_general(xk, win_ref[...], (((1,), (1,)), ((), ())),
                              precision=_HIGH) + bin_ref[...]
        h_r = lax.dot_general(h, whr_ref[...], (((1,), (1,)), ((), ())),
                              precision=_HIGH) + bhr_ref[...]
        h_z = lax.dot_general(h, whz_ref[...], (((1,), (1,)), ((), ())),
                              precision=_HIGH) + bhz_ref[...]
        h_n = lax.dot_general(h, whn_ref[...], (((1,), (1,)), ((), ())),
                              precision=_HIGH) + bhn_ref[...]
        r = jax.nn.sigmoid(i_r + h_r)
        z = jax.nn.sigmoid(i_z + h_z)
        n = jnp.tanh(i_n + r * h_n)
        hn = (1.0 - z) * n + z * h
        h = jnp.where(valid, hn, h)
    pred = lax.dot_general(h, wp_ref[...], (((1,), (0,)), ((), ())),
                           precision=_HIGH) + bp_ref[0, 0]
    out_ref[0] = pred


def kernel(x, edge_index, edge_weight, W1, b1, W2, b2,
           W_ih, W_hh, b_ih, b_hh, Wp, bp):
    f32 = jnp.float32
    src = edge_index[:, 0, :]
    dst = edge_index[:, 1, :]
    srcg = (src + (jnp.arange(T, dtype=jnp.int32) * N)[:, None]
            ).reshape(T, NW, NSUB, SUB)
    dstg = dst.reshape(T, NW, NSUB, SUB)
    wg = edge_weight.reshape(T, NW, NSUB, SUB)

    degp = _deg_kernel(dstg, wg).reshape(T, NW, N)

    b1r = b1.reshape(1, H1)
    b2r = b2.reshape(1, H1)

    y1, dis = pl.pallas_call(
        _tc1_body,
        grid=(T,),
        in_specs=[
            pl.BlockSpec((1, D, N), lambda t: (t, 0, 0)),
            pl.BlockSpec((1, NW, N), lambda t: (t, 0, 0)),
            pl.BlockSpec((D, H1), lambda t: (0, 0)),
        ],
        out_specs=[
            pl.BlockSpec((1, N, H1), lambda t: (t, 0, 0)),
            pl.BlockSpec((1, N, 1), lambda t: (t, 0, 0)),
        ],
        out_shape=[
            jax.ShapeDtypeStruct((T, N, H1), f32),
            jax.ShapeDtypeStruct((T, N, 1), f32),
        ],
        compiler_params=_tc_params,
    )(x, degp, W1)

    part1 = _edge_kernel(y1.reshape(T * N, H1), srcg, dstg, wg)

    y2 = pl.pallas_call(
        _tc2_body,
        grid=(T, NB),
        in_specs=[
            pl.BlockSpec((NC, 1, BN, H1), lambda t, b: (0, t, b, 0)),
            pl.BlockSpec((1, BN, H1), lambda t, b: (t, b, 0)),
            pl.BlockSpec((1, BN, 1), lambda t, b: (t, b, 0)),
            pl.BlockSpec((H1, H1), lambda t, b: (0, 0)),
            pl.BlockSpec((1, H1), lambda t, b: (0, 0)),
        ],
        out_specs=pl.BlockSpec((1, BN, H1), lambda t, b: (t, b, 0)),
        out_shape=jax.ShapeDtypeStruct((T, N, H1), f32),
        compiler_params=_tc_params,
    )(part1, y1, dis, W2, b1r)

    part2 = _edge_kernel(y2.reshape(T * N, H1), srcg, dstg, wg)

    g = pl.pallas_call(
        _tc3a_body,
        grid=(T, NB),
        in_specs=[
            pl.BlockSpec((NC, 1, BN, H1), lambda t, b: (0, t, b, 0)),
            pl.BlockSpec((1, BN, H1), lambda t, b: (t, b, 0)),
            pl.BlockSpec((1, BN, 1), lambda t, b: (t, b, 0)),
            pl.BlockSpec((1, H1), lambda t, b: (0, 0)),
        ],
        out_specs=pl.BlockSpec((1, BN, H1), lambda t, b: (t, b, 0)),
        out_shape=jax.ShapeDtypeStruct((T, N, H1), f32),
        compiler_params=_tc_params,
    )(part2, y2, dis, b2r)

    wir, wiz, win = jnp.split(W_ih, 3, axis=0)
    whr, whz, whn = jnp.split(W_hh, 3, axis=0)
    bir, biz, bin_ = [v.reshape(1, HG) for v in jnp.split(b_ih, 3)]
    bhr, bhz, bhn = [v.reshape(1, HG) for v in jnp.split(b_hh, 3)]
    wp_wide = jnp.pad(Wp, ((0, 0), (0, 7)))
    bpr = bp.reshape(1, 1)

    g_specs = [
        pl.BlockSpec((1, BN, H1),
                     (lambda t, b, k=k: (jnp.maximum(t - 3 + k, 0), b, 0)))
        for k in range(4)
    ]
    w_specs = (
        [pl.BlockSpec((HG, H1), lambda t, b: (0, 0))] * 3
        + [pl.BlockSpec((HG, HG), lambda t, b: (0, 0))] * 3
        + [pl.BlockSpec((1, HG), lambda t, b: (0, 0))] * 6
        + [pl.BlockSpec((HG, 8), lambda t, b: (0, 0)),
           pl.BlockSpec((1, 1), lambda t, b: (0, 0))]
    )

    preds = pl.pallas_call(
        _gru_body,
        grid=(T, NB),
        in_specs=g_specs + w_specs,
        out_specs=pl.BlockSpec((1, BN, 8), lambda t, b: (t, b, 0)),
        out_shape=jax.ShapeDtypeStruct((T, N, 8), f32),
        compiler_params=_tc_params,
    )(g, g, g, g, wir, wiz, win, whr, whz, whn,
      bir, biz, bin_, bhr, bhz, bhn, wp_wide, bpr)

    return preds[:, :, 0]


# speedup vs baseline: 15.5195x; 1.3294x over previous
import functools

import jax
import jax.numpy as jnp
import numpy as np
from jax import lax
from jax.experimental import pallas as pl
from jax.experimental.pallas import tpu as pltpu
from jax.experimental.pallas import tpu_sc as plsc

T = 8
N = 10000
E = 320000
D = 128
H1 = 64
HG = 32

NC = 2
NS = 16
NW = NC * NS
EPW = E // NW
SUB = 128
NSUB = 80
EPP = NSUB * SUB
QUAD = 4
STRIPE = 624
ZROWS = 104

_mesh = plsc.VectorSubcoreMesh(core_axis_name="c", subcore_axis_name="s")
_sc_params = pltpu.CompilerParams(needs_layout_passes=False,
                                  use_tc_tiling_on_sc=False)
_tc_params = pltpu.CompilerParams(vmem_limit_bytes=120 * 1024 * 1024)


@functools.partial(
    pl.kernel,
    mesh=_mesh,
    out_type=jax.ShapeDtypeStruct((T, NW, 1, N), jnp.float32),
    scratch_types=[
        pltpu.VMEM((1, N), jnp.float32),
        pltpu.VMEM((NSUB, SUB), jnp.int32),
        pltpu.VMEM((NSUB, SUB), jnp.float32),
    ],
    compiler_params=_sc_params,
)
def _deg_kernel(dst_hbm, w_hbm, out_hbm, deg_v, dst_v, w_v):
    c = lax.axis_index("c")
    s = lax.axis_index("s")
    gid = c * NS + s

    z16 = jnp.zeros((16,), jnp.float32)
    zsplat = jnp.zeros((16,), jnp.int32)
    lanes = lax.iota(jnp.int32, 16)
    lane_masks = [lanes == l for l in range(16)]

    def zero_body(i, carry):
        deg_v[0, pl.ds(i * 16, 16)] = z16
        return carry

    lax.fori_loop(0, N // 16, zero_body, 0)

    for t in range(T):
        pltpu.sync_copy(dst_hbm.at[t, gid], dst_v)
        pltpu.sync_copy(w_hbm.at[t, gid], w_v)

        def row_body(r, carry):
            for co in range(SUB // 16):
                d16 = dst_v[r, pl.ds(co * 16, 16)]
                w16 = w_v[r, pl.ds(co * 16, 16)]
                for l in range(16):
                    plsc.addupdate_scatter(deg_v, [zsplat, d16], w16,
                                           mask=lane_masks[l])
            return carry

        lax.fori_loop(0, NSUB, row_body, 0)

        pltpu.sync_copy(deg_v, out_hbm.at[t, gid])
        lax.fori_loop(0, N // 16, zero_body, 0)


@functools.partial(
    pl.kernel,
    mesh=_mesh,
    out_type=jax.ShapeDtypeStruct((NC, T, N, H1), jnp.float32),
    scratch_types=[
        pltpu.VMEM_SHARED((N, H1), jnp.float32),
        pltpu.VMEM((NSUB, SUB), jnp.int32),
        pltpu.VMEM((NSUB, SUB), jnp.int32),
        pltpu.VMEM((NSUB, SUB), jnp.float32),
        [pltpu.VMEM((SUB, H1), jnp.float32) for _ in range(QUAD)],
        pltpu.VMEM((ZROWS, H1), jnp.float32),
        [pltpu.SemaphoreType.DMA for _ in range(QUAD)],
        pltpu.SemaphoreType.DMA,
    ],
    compiler_params=_sc_params,
)
def _edge_kernel(y_hbm, src_hbm, dst_hbm, w_hbm, out_hbm,
                 acc, src_v, dst_v, w_v, rows_bufs, zero_v, gsems, ssem):
    c = lax.axis_index("c")
    s = lax.axis_index("s")
    gid = c * NS + s
    base = s * STRIPE
    is_last = s == (NS - 1)

    z16 = jnp.zeros((16,), jnp.float32)
    zsplat = jnp.zeros((16,), jnp.int32)

    def zfill_body(i, carry):
        zero_v[i // 4, pl.ds((i % 4) * 16, 16)] = z16
        return carry

    lax.fori_loop(0, ZROWS * 4, zfill_body, 0)

    for t in range(T):
        for k in range(STRIPE // ZROWS):
            pltpu.sync_copy(zero_v, acc.at[pl.ds(base + k * ZROWS, ZROWS)])

        @pl.when(is_last)
        def _():
            pltpu.sync_copy(zero_v.at[pl.ds(0, 16)],
                            acc.at[pl.ds(N - 16, 16)])

        plsc.subcore_barrier()

        pltpu.sync_copy(src_hbm.at[t, gid], src_v)
        pltpu.sync_copy(dst_hbm.at[t, gid], dst_v)
        pltpu.sync_copy(w_hbm.at[t, gid], w_v)

        def quad_body(q, carry):
            gds = [pltpu.async_copy(y_hbm.at[src_v.at[q * QUAD + i]],
                                    rows_bufs[i], gsems[i])
                   for i in range(QUAD)]
            sds = []
            for i in range(QUAD):
                j = q * QUAD + i
                gds[i].wait()
                rows_v = rows_bufs[i]

                def e_body(e, ecarry, j=j, rows_v=rows_v):
                    wb = plsc.load_gather(w_v, [zsplat + j, zsplat + e])
                    for rr in range(H1 // 16):
                        vals = rows_v[e, pl.ds(rr * 16, 16)]
                        rows_v[e, pl.ds(rr * 16, 16)] = vals * wb
                    return ecarry

                lax.fori_loop(0, SUB, e_body, 0)
                sds.append(pltpu.async_copy(rows_v, acc.at[dst_v.at[j]],
                                            ssem, add=True))
            for sd in sds:
                sd.wait()
            return carry

        lax.fori_loop(0, NSUB // QUAD, quad_body, 0)
        plsc.subcore_barrier()

        pltpu.sync_copy(acc.at[pl.ds(base, STRIPE)],
                        out_hbm.at[c, t, pl.ds(base, STRIPE)])

        @pl.when(is_last)
        def _():
            pltpu.sync_copy(acc.at[pl.ds(N - 16, 16)],
                            out_hbm.at[c, t, pl.ds(N - 16, 16)])


_HIGH = lax.Precision.HIGHEST
NB = 5
BN = N // NB


def _tc1_body(x_ref, dp_ref, w1_ref, y1_ref, dis_ref):
    dp = dp_ref[0]
    ones64 = jnp.ones_like(dp[:, :H1])
    deg64 = lax.dot_general(dp, ones64, (((0,), (0,)), ((), ())),
                            precision=_HIGH) + 1.0
    dis64 = lax.rsqrt(deg64)
    xw = lax.dot_general(x_ref[0], w1_ref[...], (((0,), (0,)), ((), ())),
                         precision=_HIGH)
    y1_ref[0] = xw * dis64
    ones1 = jnp.ones_like(dp[:, :1])
    deg1 = lax.dot_general(dp, ones1, (((0,), (0,)), ((), ())),
                           precision=_HIGH) + 1.0
    dis_ref[0] = lax.rsqrt(deg1)


def _tc2_body(p_ref, y1_ref, dis_ref, w2_ref, b1_ref, y2_ref):
    S = p_ref[0, 0] + p_ref[1, 0]
    dis = dis_ref[0]
    h1 = jnp.maximum(dis * (S + y1_ref[0]) + b1_ref[...], 0.0)
    xw2 = lax.dot_general(h1, w2_ref[...], (((1,), (0,)), ((), ())),
                          precision=_HIGH)
    y2_ref[0] = xw2 * dis


def _tc3a_body(p_ref, y2_ref, dis_ref, b2_ref, g_ref):
    S = p_ref[0, 0] + p_ref[1, 0]
    dis = dis_ref[0]
    g_ref[0] = jnp.maximum(dis * (S + y2_ref[0]) + b2_ref[...], 0.0)


def _gru_body(g0_ref, g1_ref, g2_ref, g3_ref,
              wir_ref, wiz_ref, win_ref, whr_ref, whz_ref, whn_ref,
              bir_ref, biz_ref, bin_ref, bhr_ref, bhz_ref, bhn_ref,
              wp_ref, bp_ref, out_ref):
    t = pl.program_id(0)
    g_refs = (g0_ref, g1_ref, g2_ref, g3_ref)
    h = jnp.zeros_like(g0_ref[0][:, :HG])
    for k in range(4):
        valid = (t - 3 + k) >= 0
        xk = g_refs[k][0]
        i_r = lax.dot_general(xk, wir_ref[...], (((1,), (1,)), ((), ())),
                              precision=_HIGH) + bir_ref[...]
        i_z = lax.dot_general(xk, wiz_ref[...], (((1,), (1,)), ((), ())),
                              precision=_HIGH) + biz_ref[...]
        i_n = lax.dot_general(xk, win_ref[...], (((1,), (1,)), ((), ())),
                              precision=_HIGH) + bin_ref[...]
        h_r = lax.dot_general(h, whr_ref[...], (((1,), (1,)), ((), ())),
                              precision=_HIGH) + bhr_ref[...]
        h_z = lax.dot_general(h, whz_ref[...], (((1,), (1,)), ((), ())),
                              precision=_HIGH) + bhz_ref[...]
        h_n = lax.dot_general(h, whn_ref[...], (((1,), (1,)), ((), ())),
                              precision=_HIGH) + bhn_ref[...]
        r = jax.nn.sigmoid(i_r + h_r)
        z = jax.nn.sigmoid(i_z + h_z)
        n = jnp.tanh(i_n + r * h_n)
        hn = (1.0 - z) * n + z * h
        h = jnp.where(valid, hn, h)
    pred = lax.dot_general(h, wp_ref[...], (((1,), (0,)), ((), ())),
                           precision=_HIGH) + bp_ref[0, 0]
    out_ref[0] = pred


def kernel(x, edge_index, edge_weight, W1, b1, W2, b2,
           W_ih, W_hh, b_ih, b_hh, Wp, bp):
    f32 = jnp.float32
    src = edge_index[:, 0, :].reshape(T, NW, EPW)
    dst = edge_index[:, 1, :].reshape(T, NW, EPW)
    npad = EPP - EPW
    pad_ids = ((jnp.arange(npad, dtype=jnp.int32)[None, :]
                + jnp.arange(NW, dtype=jnp.int32)[:, None] * npad) % N)
    pad_blk = jnp.broadcast_to(pad_ids[None], (T, NW, npad))
    shift = (jnp.arange(T, dtype=jnp.int32) * N)[:, None, None]
    srcg = (jnp.concatenate([src, pad_blk], axis=2) + shift
            ).reshape(T, NW, NSUB, SUB)
    dstg = jnp.concatenate([dst, pad_blk], axis=2).reshape(T, NW, NSUB, SUB)
    wg = jnp.concatenate(
        [edge_weight.reshape(T, NW, EPW),
         jnp.zeros((T, NW, npad), jnp.float32)], axis=2,
    ).reshape(T, NW, NSUB, SUB)

    degp = _deg_kernel(dstg, wg).reshape(T, NW, N)

    b1r = b1.reshape(1, H1)
    b2r = b2.reshape(1, H1)

    y1, dis = pl.pallas_call(
        _tc1_body,
        grid=(T,),
        in_specs=[
            pl.BlockSpec((1, D, N), lambda t: (t, 0, 0)),
            pl.BlockSpec((1, NW, N), lambda t: (t, 0, 0)),
            pl.BlockSpec((D, H1), lambda t: (0, 0)),
        ],
        out_specs=[
            pl.BlockSpec((1, N, H1), lambda t: (t, 0, 0)),
            pl.BlockSpec((1, N, 1), lambda t: (t, 0, 0)),
        ],
        out_shape=[
            jax.ShapeDtypeStruct((T, N, H1), f32),
            jax.ShapeDtypeStruct((T, N, 1), f32),
        ],
        compiler_params=_tc_params,
    )(x, degp, W1)

    part1 = _edge_kernel(y1.reshape(T * N, H1), srcg, dstg, wg)

    y2 = pl.pallas_call(
        _tc2_body,
        grid=(T, NB),
        in_specs=[
            pl.BlockSpec((NC, 1, BN, H1), lambda t, b: (0, t, b, 0)),
            pl.BlockSpec((1, BN, H1), lambda t, b: (t, b, 0)),
            pl.BlockSpec((1, BN, 1), lambda t, b: (t, b, 0)),
            pl.BlockSpec((H1, H1), lambda t, b: (0, 0)),
            pl.BlockSpec((1, H1), lambda t, b: (0, 0)),
        ],
        out_specs=pl.BlockSpec((1, BN, H1), lambda t, b: (t, b, 0)),
        out_shape=jax.ShapeDtypeStruct((T, N, H1), f32),
        compiler_params=_tc_params,
    )(part1, y1, dis, W2, b1r)

    part2 = _edge_kernel(y2.reshape(T * N, H1), srcg, dstg, wg)

    g = pl.pallas_call(
        _tc3a_body,
        grid=(T, NB),
        in_specs=[
            pl.BlockSpec((NC, 1, BN, H1), lambda t, b: (0, t, b, 0)),
            pl.BlockSpec((1, BN, H1), lambda t, b: (t, b, 0)),
            pl.BlockSpec((1, BN, 1), lambda t, b: (t, b, 0)),
            pl.BlockSpec((1, H1), lambda t, b: (0, 0)),
        ],
        out_specs=pl.BlockSpec((1, BN, H1), lambda t, b: (t, b, 0)),
        out_shape=jax.ShapeDtypeStruct((T, N, H1), f32),
        compiler_params=_tc_params,
    )(part2, y2, dis, b2r)

    wir, wiz, win = jnp.split(W_ih, 3, axis=0)
    whr, whz, whn = jnp.split(W_hh, 3, axis=0)
    bir, biz, bin_ = [v.reshape(1, HG) for v in jnp.split(b_ih, 3)]
    bhr, bhz, bhn = [v.reshape(1, HG) for v in jnp.split(b_hh, 3)]
    wp_wide = jnp.pad(Wp, ((0, 0), (0, 7)))
    bpr = bp.reshape(1, 1)

    g_specs = [
        pl.BlockSpec((1, BN, H1),
                     (lambda t, b, k=k: (jnp.maximum(t - 3 + k, 0), b, 0)))
        for k in range(4)
    ]
    w_specs = (
        [pl.BlockSpec((HG, H1), lambda t, b: (0, 0))] * 3
        + [pl.BlockSpec((HG, HG), lambda t, b: (0, 0))] * 3
        + [pl.BlockSpec((1, HG), lambda t, b: (0, 0))] * 6
        + [pl.BlockSpec((HG, 8), lambda t, b: (0, 0)),
           pl.BlockSpec((1, 1), lambda t, b: (0, 0))]
    )

    preds = pl.pallas_call(
        _gru_body,
        grid=(T, NB),
        in_specs=g_specs + w_specs,
        out_specs=pl.BlockSpec((1, BN, 8), lambda t, b: (t, b, 0)),
        out_shape=jax.ShapeDtypeStruct((T, N, 8), f32),
        compiler_params=_tc_params,
    )(g, g, g, g, wir, wiz, win, whr, whz, whn,
      bir, biz, bin_, bhr, bhz, bhn, wp_wide, bpr)

    return preds[:, :, 0]
